# baseline (speedup 1.0000x reference)
import jax
import jax.numpy as jnp
from jax.experimental import pallas as pl
from jax.experimental.pallas import tpu as pltpu

T = 16384
D_MODEL = 768
D_FEAT = 64
D_RH = 64
D_EH = 64
N_EXP = 8
D_E1 = N_EXP * D_EH

BT = 2048


def _moe_block(hid_ref, feat_ref, w1a_ref, w1b_ref, b1_ref, wr1a_ref,
               wr1b_ref, br1_ref, wr2_ref, br2_ref, expand_ref, w2_ref,
               be2_ref, out_hbm, obuf, osem):
    i = pl.program_id(0)
    slot = jax.lax.rem(i, 2)

    def out_copy(j, s):
        return pltpu.make_async_copy(
            obuf.at[s], out_hbm.at[pl.ds(j * BT, BT), :], osem.at[s])

    hid = hid_ref[...]
    feat = feat_ref[...]

    h = jnp.dot(hid.astype(jnp.bfloat16), w1a_ref[...],
                preferred_element_type=jnp.float32)
    h += jnp.dot(feat.astype(jnp.bfloat16), w1b_ref[...],
                 preferred_element_type=jnp.float32)
    h = jax.nn.gelu(h + b1_ref[...])

    hr = jnp.dot(hid, wr1a_ref[...], preferred_element_type=jnp.float32)
    hr += jnp.dot(feat, wr1b_ref[...], preferred_element_type=jnp.float32)
    hr = jax.nn.gelu(hr + br1_ref[...])
    logits = jnp.dot(hr, wr2_ref[...],
                     preferred_element_type=jnp.float32) + br2_ref[...]
    col = jax.lax.broadcasted_iota(jnp.int32, logits.shape, 1)
    i1 = jnp.argmax(logits, axis=-1, keepdims=True)
    m1 = jnp.max(logits, axis=-1, keepdims=True)
    oh1 = (col == i1).astype(jnp.float32)
    masked = jnp.where(col == i1, -jnp.inf, logits)
    i2 = jnp.argmax(masked, axis=-1, keepdims=True)
    m2 = jnp.max(masked, axis=-1, keepdims=True)
    oh2 = (col == i2).astype(jnp.float32)
    s2 = 1.0 / (1.0 + jnp.exp(m1 - m2))
    w8 = oh1 * (1.0 - s2) + oh2 * s2

    wexp = jnp.dot(w8, expand_ref[...],
                   preferred_element_type=jnp.float32).astype(jnp.bfloat16)
    hw = h.astype(jnp.bfloat16) * wexp
    y = hid + (
        jnp.dot(hw, w2_ref[...], preferred_element_type=jnp.float32)
        + jnp.dot(w8, be2_ref[...], preferred_element_type=jnp.float32))

    @pl.when(i >= 2)
    def _():
        out_copy(i - 2, slot).wait()

    obuf[slot] = y
    out_copy(i, slot).start()

    @pl.when(i == (T // BT) - 1)
    def _():
        out_copy(i - 1, 1 - slot).wait()
        out_copy(i, slot).wait()


@jax.jit
def kernel(hidden, feat, w_r1, b_r1, w_r2, b_r2, w_e1, b_e1, w_e2, b_e2):
    d_in = D_MODEL + D_FEAT
    w1 = w_e1.transpose(1, 0, 2).reshape(d_in, D_E1).astype(jnp.bfloat16)
    w1a = w1[:D_MODEL]
    w1b = w1[D_MODEL:]
    b1 = b_e1.reshape(1, D_E1)
    wr1a = w_r1[:D_MODEL]
    wr1b = w_r1[D_MODEL:]
    br1 = b_r1.reshape(1, -1)
    br2 = b_r2.reshape(1, -1)
    w2 = w_e2.reshape(D_E1, D_MODEL).astype(jnp.bfloat16)
    expand = jnp.repeat(jnp.eye(N_EXP, dtype=jnp.float32), D_EH, axis=1)

    out = pl.pallas_call(
        _moe_block,
        grid=(T // BT,),
        in_specs=[
            pl.BlockSpec((BT, D_MODEL), lambda i: (i, 0)),
            pl.BlockSpec((BT, D_FEAT), lambda i: (i, 0)),
            pl.BlockSpec((D_MODEL, D_E1), lambda i: (0, 0)),
            pl.BlockSpec((D_FEAT, D_E1), lambda i: (0, 0)),
            pl.BlockSpec((1, D_E1), lambda i: (0, 0)),
            pl.BlockSpec((D_MODEL, D_RH), lambda i: (0, 0)),
            pl.BlockSpec((D_FEAT, D_RH), lambda i: (0, 0)),
            pl.BlockSpec((1, D_RH), lambda i: (0, 0)),
            pl.BlockSpec((D_RH, N_EXP), lambda i: (0, 0)),
            pl.BlockSpec((1, N_EXP), lambda i: (0, 0)),
            pl.BlockSpec((N_EXP, D_E1), lambda i: (0, 0)),
            pl.BlockSpec((D_E1, D_MODEL), lambda i: (0, 0)),
            pl.BlockSpec((N_EXP, D_MODEL), lambda i: (0, 0)),
        ],
        out_specs=pl.BlockSpec(memory_space=pl.ANY),
        out_shape=jax.ShapeDtypeStruct((T, D_MODEL), jnp.float32),
        scratch_shapes=[
            pltpu.VMEM((2, BT, D_MODEL), jnp.float32),
            pltpu.SemaphoreType.DMA((2,)),
        ],
        compiler_params=pltpu.CompilerParams(
            dimension_semantics=("arbitrary",)),
    )(hidden, feat, w1a, w1b, b1, wr1a, wr1b, br1, w_r2, br2, expand, w2,
      b_e2)
    return out


# speedup vs baseline: 1.0359x; 1.0359x over previous
import jax
import jax.numpy as jnp
from jax.experimental import pallas as pl
from jax.experimental.pallas import tpu as pltpu

T = 16384
D_MODEL = 768
D_FEAT = 64
D_RH = 64
D_EH = 64
N_EXP = 8
D_E1 = N_EXP * D_EH

BT = 2048


def _moe_block(hid_ref, feat_ref, w1a_ref, w1b_ref, b1_ref, wr1a_ref,
               wr1b_ref, br1_ref, wr2_ref, br2_ref, expand_ref, w2_ref,
               be2_ref, out_ref):
    hid = hid_ref[...]
    feat = feat_ref[...]

    h = jnp.dot(hid.astype(jnp.bfloat16), w1a_ref[...],
                preferred_element_type=jnp.float32)
    h += jnp.dot(feat.astype(jnp.bfloat16), w1b_ref[...],
                 preferred_element_type=jnp.float32)
    h = jax.nn.gelu(h + b1_ref[...])

    hr = jnp.dot(hid, wr1a_ref[...], preferred_element_type=jnp.float32)
    hr += jnp.dot(feat, wr1b_ref[...], preferred_element_type=jnp.float32)
    hr = jax.nn.gelu(hr + br1_ref[...])
    logits = jnp.dot(hr, wr2_ref[...],
                     preferred_element_type=jnp.float32) + br2_ref[...]
    col = jax.lax.broadcasted_iota(jnp.int32, logits.shape, 1)
    i1 = jnp.argmax(logits, axis=-1, keepdims=True)
    m1 = jnp.max(logits, axis=-1, keepdims=True)
    oh1 = (col == i1).astype(jnp.float32)
    masked = jnp.where(col == i1, -jnp.inf, logits)
    i2 = jnp.argmax(masked, axis=-1, keepdims=True)
    m2 = jnp.max(masked, axis=-1, keepdims=True)
    oh2 = (col == i2).astype(jnp.float32)
    s2 = 1.0 / (1.0 + jnp.exp(m1 - m2))
    w8 = oh1 * (1.0 - s2) + oh2 * s2

    wexp = jnp.dot(w8, expand_ref[...],
                   preferred_element_type=jnp.float32).astype(jnp.bfloat16)
    hw = h.astype(jnp.bfloat16) * wexp
    out_ref[...] = hid + (
        jnp.dot(hw, w2_ref[...], preferred_element_type=jnp.float32)
        + jnp.dot(w8, be2_ref[...], preferred_element_type=jnp.float32))


@jax.jit
def kernel(hidden, feat, w_r1, b_r1, w_r2, b_r2, w_e1, b_e1, w_e2, b_e2):
    d_in = D_MODEL + D_FEAT
    w1 = w_e1.transpose(1, 0, 2).reshape(d_in, D_E1).astype(jnp.bfloat16)
    w1a = w1[:D_MODEL]
    w1b = w1[D_MODEL:]
    b1 = b_e1.reshape(1, D_E1)
    wr1a = w_r1[:D_MODEL]
    wr1b = w_r1[D_MODEL:]
    br1 = b_r1.reshape(1, -1)
    br2 = b_r2.reshape(1, -1)
    w2 = w_e2.reshape(D_E1, D_MODEL).astype(jnp.bfloat16)
    expand = jnp.repeat(jnp.eye(N_EXP, dtype=jnp.float32), D_EH, axis=1)

    out = pl.pallas_call(
        _moe_block,
        grid=(T // BT,),
        in_specs=[
            pl.BlockSpec((BT, D_MODEL), lambda i: (i, 0)),
            pl.BlockSpec((BT, D_FEAT), lambda i: (i, 0)),
            pl.BlockSpec((D_MODEL, D_E1), lambda i: (0, 0)),
            pl.BlockSpec((D_FEAT, D_E1), lambda i: (0, 0)),
            pl.BlockSpec((1, D_E1), lambda i: (0, 0)),
            pl.BlockSpec((D_MODEL, D_RH), lambda i: (0, 0)),
            pl.BlockSpec((D_FEAT, D_RH), lambda i: (0, 0)),
            pl.BlockSpec((1, D_RH), lambda i: (0, 0)),
            pl.BlockSpec((D_RH, N_EXP), lambda i: (0, 0)),
            pl.BlockSpec((1, N_EXP), lambda i: (0, 0)),
            pl.BlockSpec((N_EXP, D_E1), lambda i: (0, 0)),
            pl.BlockSpec((D_E1, D_MODEL), lambda i: (0, 0)),
            pl.BlockSpec((N_EXP, D_MODEL), lambda i: (0, 0)),
        ],
        out_specs=pl.BlockSpec((BT, D_MODEL), lambda i: (i, 0)),
        out_shape=jax.ShapeDtypeStruct((T, D_MODEL), jnp.float32),
        compiler_params=pltpu.CompilerParams(
            dimension_semantics=("parallel",)),
    )(hidden, feat, w1a, w1b, b1, wr1a, wr1b, br1, w_r2, br2, expand, w2,
      b_e2)
    return out
